# baseline (speedup 1.0000x reference)
import dataclasses
import functools

import jax
import jax.numpy as jnp
from jax import lax
from jax.experimental import pallas as pl
from jax.experimental.pallas import tpu as pltpu
from jax.experimental.pallas import tpu_sc as plsc

N_NODES = 10000
N_EDGES = 320000
HIDDEN = 128
NUM_EXPERTS = 8
MAX_ELEM = 90
NUM_GAUSS = 256
BATCH = 64

BLK = 2560
NSTEPS = N_EDGES // BLK

_DELTA = 8.0 / (NUM_GAUSS - 1)
_COEFF = -0.5 / (_DELTA * _DELTA)


_NW = 32
_PER_W = (2 * N_EDGES) // _NW


def _sc_gather_z(atomic_numbers, flat_idx):
    mesh = plsc.VectorSubcoreMesh(core_axis_name="c", subcore_axis_name="s")
    cp = pltpu.CompilerParams()
    if "needs_layout_passes" in pltpu.CompilerParams.__dataclass_fields__:
        cp = dataclasses.replace(cp, needs_layout_passes=False)

    @functools.partial(
        pl.kernel,
        mesh=mesh,
        compiler_params=cp,
        out_type=jax.ShapeDtypeStruct((2 * N_EDGES,), jnp.int32),
        scratch_types=[
            pltpu.VMEM((N_NODES,), jnp.int32),
            pltpu.VMEM((_PER_W,), jnp.int32),
            pltpu.VMEM((_PER_W,), jnp.int32),
        ],
    )
    def gather_kernel(tab_hbm, idx_hbm, out_hbm, tab_v, idx_v, out_v):
        wid = lax.axis_index("s") * 2 + lax.axis_index("c")
        base = wid * _PER_W
        pltpu.sync_copy(tab_hbm, tab_v)
        pltpu.sync_copy(idx_hbm.at[pl.ds(base, _PER_W)], idx_v)

        @pl.loop(0, _PER_W, step=16)
        def _(i):
            idx = idx_v[pl.ds(i, 16)]
            out_v[pl.ds(i, 16)] = plsc.load_gather(tab_v, [idx])

        pltpu.sync_copy(out_v, out_hbm.at[pl.ds(base, _PER_W)])

    return gather_kernel(atomic_numbers, flat_idx)


def _tc_body(d_ref, zs_ref, zt_ref, ti_ref, batch_ref, offs_ref, srange_ref,
             wf_ref, we_ref, wp_ref, bd_ref, be_ref, bp_ref,
             w1p_ref, b1p_ref, w2p_ref, b2p_ref,
             out_ref, acc_ref, cnt_ref, bounds_ref):
    i = pl.program_id(0)

    @pl.when(i == 0)
    def _():
        acc_ref[...] = jnp.zeros_like(acc_ref)
        cnt_ref[...] = jnp.zeros_like(cnt_ref)
        b = batch_ref[...]
        lanes = lax.broadcasted_iota(jnp.int32, (1, BATCH), 1)
        starts = jnp.sum((b < lanes).astype(jnp.int32), axis=0, keepdims=True)
        ends = jnp.sum((b <= lanes).astype(jnp.int32), axis=0, keepdims=True)
        starts_c = jnp.transpose(
            jnp.broadcast_to(starts, (BATCH, BATCH)))[:, 0:1]
        ends_c = jnp.transpose(jnp.broadcast_to(ends, (BATCH, BATCH)))[:, 0:1]
        bounds_ref[:, 0:1] = starts_c
        bounds_ref[:, 1:2] = ends_c

    d = d_ref[0]
    offs = offs_ref[...]
    diff = d - offs
    arg = (_COEFF * diff) * diff
    gauss = jnp.exp(arg).astype(jnp.bfloat16)

    srange = srange_ref[...]
    ohs = (zs_ref[0] == srange).astype(jnp.bfloat16)
    oht = (zt_ref[0] == srange).astype(jnp.bfloat16)

    cat = jnp.concatenate([gauss, ohs, oht], axis=0)
    x = jnp.dot(wf_ref[...], cat, preferred_element_type=jnp.float32)
    x = jax.nn.silu(x + bd_ref[...]).astype(jnp.bfloat16)
    x = jnp.dot(we_ref[...], x, preferred_element_type=jnp.float32)
    x = jax.nn.silu(x + be_ref[...]).astype(jnp.bfloat16)
    x = jnp.dot(wp_ref[...], x, preferred_element_type=jnp.float32)
    x = jax.nn.silu(x + bp_ref[...]).astype(jnp.bfloat16)

    ti = ti_ref[0]
    starts_c = bounds_ref[:, 0:1]
    ends_c = bounds_ref[:, 1:2]
    seg = jnp.logical_and(ti >= starts_c, ti < ends_c)
    segb = seg.astype(jnp.bfloat16)

    acc_ref[...] += lax.dot_general(
        segb, x, (((1,), (1,)), ((), ())), preferred_element_type=jnp.float32)
    cnt_ref[...] += jnp.sum(seg.astype(jnp.float32), axis=1, keepdims=True)

    @pl.when(i == NSTEPS - 1)
    def _():
        xg = acc_ref[...] / (cnt_ref[...] + 0.001)
        h = jnp.dot(xg.astype(jnp.bfloat16), w1p_ref[...],
                    preferred_element_type=jnp.float32)
        h = jax.nn.silu(h + b1p_ref[...])
        out = jnp.dot(h.astype(jnp.bfloat16), w2p_ref[...],
                      preferred_element_type=jnp.float32)
        out_ref[...] = out + b2p_ref[...]


def _row_spec():
    return pl.BlockSpec((1, 1, BLK), lambda i: (i, 0, 0))


def _full_spec(shape):
    return pl.BlockSpec(shape, lambda i: tuple(0 for _ in shape))


def kernel(atomic_numbers, edge_distance, edge_index, batch, batch_size,
           source_emb, target_emb, W_dist, b_dist, W_edge, b_edge,
           W1_pre, b1_pre, W1_post, b1_post, W2_post, b2_post):
    flat_idx = edge_index.reshape(2 * N_EDGES)
    zz = _sc_gather_z(atomic_numbers, flat_idx)
    zs = zz[:N_EDGES].reshape(NSTEPS, 1, BLK)
    zt = zz[N_EDGES:].reshape(NSTEPS, 1, BLK)

    d = edge_distance.reshape(NSTEPS, 1, BLK)
    ti = edge_index[1].reshape(NSTEPS, 1, BLK)
    b2d = batch.reshape(N_NODES, 1)

    offs = jnp.linspace(0.0, 8.0, NUM_GAUSS).reshape(NUM_GAUSS, 1)
    srange = jnp.arange(HIDDEN, dtype=jnp.int32).reshape(HIDDEN, 1)

    pad = jnp.zeros((HIDDEN - MAX_ELEM, HIDDEN), jnp.float32)
    w_fused = jnp.concatenate(
        [W_dist, source_emb, pad, target_emb, pad],
        axis=0).T.astype(jnp.bfloat16)
    we = W_edge.T.astype(jnp.bfloat16)
    wp = W1_pre.T.astype(jnp.bfloat16)
    w1p = W1_post.astype(jnp.bfloat16)
    w2p = W2_post.astype(jnp.bfloat16)
    bd = b_dist.reshape(HIDDEN, 1)
    be = b_edge.reshape(HIDDEN, 1)
    bp = b1_pre.reshape(HIDDEN, 1)
    b1p = b1_post.reshape(1, HIDDEN)
    b2p = b2_post.reshape(1, NUM_EXPERTS)

    out = pl.pallas_call(
        _tc_body,
        grid=(NSTEPS,),
        in_specs=[
            _row_spec(),
            _row_spec(),
            _row_spec(),
            _row_spec(),
            _full_spec((N_NODES, 1)),
            _full_spec((NUM_GAUSS, 1)),
            _full_spec((HIDDEN, 1)),
            _full_spec((HIDDEN, NUM_GAUSS + 2 * HIDDEN)),
            _full_spec((HIDDEN, HIDDEN)),
            _full_spec((HIDDEN, HIDDEN)),
            _full_spec((HIDDEN, 1)),
            _full_spec((HIDDEN, 1)),
            _full_spec((HIDDEN, 1)),
            _full_spec((HIDDEN, HIDDEN)),
            _full_spec((1, HIDDEN)),
            _full_spec((HIDDEN, NUM_EXPERTS)),
            _full_spec((1, NUM_EXPERTS)),
        ],
        out_specs=_full_spec((BATCH, NUM_EXPERTS)),
        out_shape=jax.ShapeDtypeStruct((BATCH, NUM_EXPERTS), jnp.float32),
        scratch_shapes=[
            pltpu.VMEM((BATCH, HIDDEN), jnp.float32),
            pltpu.VMEM((BATCH, 1), jnp.float32),
            pltpu.VMEM((BATCH, 8), jnp.int32),
        ],
        compiler_params=pltpu.CompilerParams(
            dimension_semantics=("arbitrary",)),
    )(d, zs, zt, ti, b2d, offs, srange, w_fused, we, wp, bd, be, bp,
      w1p, b1p, w2p, b2p)
    return out


# speedup vs baseline: 1.0031x; 1.0031x over previous
import dataclasses
import functools

import jax
import jax.numpy as jnp
from jax import lax
from jax.experimental import pallas as pl
from jax.experimental.pallas import tpu as pltpu
from jax.experimental.pallas import tpu_sc as plsc

N_NODES = 10000
N_EDGES = 320000
HIDDEN = 128
NUM_EXPERTS = 8
MAX_ELEM = 90
NUM_GAUSS = 256
BATCH = 64

BLK = 2560
NSTEPS = N_EDGES // BLK

_DELTA = 8.0 / (NUM_GAUSS - 1)
_COEFF = -0.5 / (_DELTA * _DELTA)
_LOG2E = 1.4426950408889634
_COEFF2 = _COEFF * _LOG2E


def _recip(v):
    if hasattr(pltpu, "reciprocal"):
        return pltpu.reciprocal(v, approx=True)
    return 1.0 / v


def _silu(x):
    e = jnp.exp2(x * (-_LOG2E))
    return (x * _recip(1.0 + e)).astype(jnp.bfloat16)


_NW = 32
_PER_W = (2 * N_EDGES) // _NW


def _sc_gather_z(atomic_numbers, flat_idx):
    mesh = plsc.VectorSubcoreMesh(core_axis_name="c", subcore_axis_name="s")
    cp = pltpu.CompilerParams()
    if "needs_layout_passes" in pltpu.CompilerParams.__dataclass_fields__:
        cp = dataclasses.replace(cp, needs_layout_passes=False)

    @functools.partial(
        pl.kernel,
        mesh=mesh,
        compiler_params=cp,
        out_type=jax.ShapeDtypeStruct((2 * N_EDGES,), jnp.int32),
        scratch_types=[
            pltpu.VMEM((N_NODES,), jnp.int32),
            pltpu.VMEM((_PER_W,), jnp.int32),
            pltpu.VMEM((_PER_W,), jnp.int32),
        ],
    )
    def gather_kernel(tab_hbm, idx_hbm, out_hbm, tab_v, idx_v, out_v):
        wid = lax.axis_index("s") * 2 + lax.axis_index("c")
        base = wid * _PER_W
        pltpu.sync_copy(tab_hbm, tab_v)
        pltpu.sync_copy(idx_hbm.at[pl.ds(base, _PER_W)], idx_v)

        @pl.loop(0, _PER_W, step=16)
        def _(i):
            idx = idx_v[pl.ds(i, 16)]
            out_v[pl.ds(i, 16)] = plsc.load_gather(tab_v, [idx])

        pltpu.sync_copy(out_v, out_hbm.at[pl.ds(base, _PER_W)])

    return gather_kernel(atomic_numbers, flat_idx)


def _tc_body(d_ref, zs_ref, zt_ref, ti_ref, batch_ref, offs_ref, srange_ref,
             wf_ref, we_ref, wp_ref, bd_ref, be_ref, bp_ref,
             w1p_ref, b1p_ref, w2p_ref, b2p_ref,
             out_ref, acc_ref, cnt_ref, bounds_ref):
    i = pl.program_id(0)

    @pl.when(i == 0)
    def _():
        acc_ref[...] = jnp.zeros_like(acc_ref)
        cnt_ref[...] = jnp.zeros_like(cnt_ref)
        b = batch_ref[...]
        lanes = lax.broadcasted_iota(jnp.int32, (1, BATCH), 1)
        starts = jnp.sum((b < lanes).astype(jnp.int32), axis=0, keepdims=True)
        ends = jnp.sum((b <= lanes).astype(jnp.int32), axis=0, keepdims=True)
        starts_c = jnp.transpose(
            jnp.broadcast_to(starts, (BATCH, BATCH)))[:, 0:1]
        ends_c = jnp.transpose(jnp.broadcast_to(ends, (BATCH, BATCH)))[:, 0:1]
        bounds_ref[:, 0:1] = starts_c
        bounds_ref[:, 1:2] = ends_c

    d = d_ref[0]
    offs = offs_ref[...]
    diff = d - offs
    arg = (_COEFF2 * diff) * diff
    gauss = jnp.exp2(arg).astype(jnp.bfloat16)

    srange = srange_ref[...]
    ohs = (zs_ref[0] == srange).astype(jnp.bfloat16)
    oht = (zt_ref[0] == srange).astype(jnp.bfloat16)

    wf = wf_ref[...]
    x = (jnp.dot(wf[:, :NUM_GAUSS], gauss,
                 preferred_element_type=jnp.float32)
         + jnp.dot(wf[:, NUM_GAUSS:NUM_GAUSS + HIDDEN], ohs,
                   preferred_element_type=jnp.float32)
         + jnp.dot(wf[:, NUM_GAUSS + HIDDEN:], oht,
                   preferred_element_type=jnp.float32))
    x = _silu(x + bd_ref[...])
    x = jnp.dot(we_ref[...], x, preferred_element_type=jnp.float32)
    x = _silu(x + be_ref[...])
    x = jnp.dot(wp_ref[...], x, preferred_element_type=jnp.float32)
    x = _silu(x + bp_ref[...])

    ti = ti_ref[0]
    starts_c = bounds_ref[:, 0:1]
    ends_c = bounds_ref[:, 1:2]
    seg = jnp.logical_and(ti >= starts_c, ti < ends_c)
    segb = seg.astype(jnp.bfloat16)

    acc_ref[...] += lax.dot_general(
        x, segb, (((1,), (1,)), ((), ())), preferred_element_type=jnp.float32)
    ones_row = jnp.ones((1, BLK), jnp.bfloat16)
    cnt_ref[...] += lax.dot_general(
        ones_row, segb, (((1,), (1,)), ((), ())),
        preferred_element_type=jnp.float32)

    @pl.when(i == NSTEPS - 1)
    def _():
        xgt = acc_ref[...] / (cnt_ref[...] + 0.001)
        h = jnp.dot(w1p_ref[...], xgt.astype(jnp.bfloat16),
                    preferred_element_type=jnp.float32)
        h = _silu(h + b1p_ref[...])
        out = jnp.dot(w2p_ref[...], h,
                      preferred_element_type=jnp.float32)
        out = out + b2p_ref[...]
        out_ref[...] = jnp.transpose(out)


def _row_spec():
    return pl.BlockSpec((1, 1, BLK), lambda i: (i, 0, 0))


def _full_spec(shape):
    return pl.BlockSpec(shape, lambda i: tuple(0 for _ in shape))


def kernel(atomic_numbers, edge_distance, edge_index, batch, batch_size,
           source_emb, target_emb, W_dist, b_dist, W_edge, b_edge,
           W1_pre, b1_pre, W1_post, b1_post, W2_post, b2_post):
    flat_idx = edge_index.reshape(2 * N_EDGES)
    zz = _sc_gather_z(atomic_numbers, flat_idx)
    zs = zz[:N_EDGES].reshape(NSTEPS, 1, BLK)
    zt = zz[N_EDGES:].reshape(NSTEPS, 1, BLK)

    d = edge_distance.reshape(NSTEPS, 1, BLK)
    ti = edge_index[1].reshape(NSTEPS, 1, BLK)
    b2d = batch.reshape(N_NODES, 1)

    offs = jnp.linspace(0.0, 8.0, NUM_GAUSS).reshape(NUM_GAUSS, 1)
    srange = jnp.arange(HIDDEN, dtype=jnp.int32).reshape(HIDDEN, 1)

    pad = jnp.zeros((HIDDEN - MAX_ELEM, HIDDEN), jnp.float32)
    w_fused = jnp.concatenate(
        [W_dist, source_emb, pad, target_emb, pad],
        axis=0).T.astype(jnp.bfloat16)
    we = W_edge.T.astype(jnp.bfloat16)
    wp = W1_pre.T.astype(jnp.bfloat16)
    w1p = W1_post.T.astype(jnp.bfloat16)
    w2p = W2_post.T.astype(jnp.bfloat16)
    bd = b_dist.reshape(HIDDEN, 1)
    be = b_edge.reshape(HIDDEN, 1)
    bp = b1_pre.reshape(HIDDEN, 1)
    b1p = b1_post.reshape(HIDDEN, 1)
    b2p = b2_post.reshape(NUM_EXPERTS, 1)

    out = pl.pallas_call(
        _tc_body,
        grid=(NSTEPS,),
        in_specs=[
            _row_spec(),
            _row_spec(),
            _row_spec(),
            _row_spec(),
            _full_spec((N_NODES, 1)),
            _full_spec((NUM_GAUSS, 1)),
            _full_spec((HIDDEN, 1)),
            _full_spec((HIDDEN, NUM_GAUSS + 2 * HIDDEN)),
            _full_spec((HIDDEN, HIDDEN)),
            _full_spec((HIDDEN, HIDDEN)),
            _full_spec((HIDDEN, 1)),
            _full_spec((HIDDEN, 1)),
            _full_spec((HIDDEN, 1)),
            _full_spec((HIDDEN, HIDDEN)),
            _full_spec((HIDDEN, 1)),
            _full_spec((NUM_EXPERTS, HIDDEN)),
            _full_spec((NUM_EXPERTS, 1)),
        ],
        out_specs=_full_spec((BATCH, NUM_EXPERTS)),
        out_shape=jax.ShapeDtypeStruct((BATCH, NUM_EXPERTS), jnp.float32),
        scratch_shapes=[
            pltpu.VMEM((HIDDEN, BATCH), jnp.float32),
            pltpu.VMEM((1, BATCH), jnp.float32),
            pltpu.VMEM((BATCH, 8), jnp.int32),
        ],
        compiler_params=pltpu.CompilerParams(
            dimension_semantics=("arbitrary",)),
    )(d, zs, zt, ti, b2d, offs, srange, w_fused, we, wp, bd, be, bp,
      w1p, b1p, w2p, b2p)
    return out


# speedup vs baseline: 1.1608x; 1.1572x over previous
import dataclasses
import functools

import jax
import jax.numpy as jnp
from jax import lax
from jax.experimental import pallas as pl
from jax.experimental.pallas import tpu as pltpu
from jax.experimental.pallas import tpu_sc as plsc

N_NODES = 10000
N_EDGES = 320000
HIDDEN = 128
NUM_EXPERTS = 8
MAX_ELEM = 90
NUM_GAUSS = 256
BATCH = 64
OH = 96

BLK = 6400
NSTEPS = N_EDGES // BLK

_DELTA = 8.0 / (NUM_GAUSS - 1)
_COEFF = -0.5 / (_DELTA * _DELTA)
_LOG2E = 1.4426950408889634
_COEFF2 = _COEFF * _LOG2E


def _recip(v):
    if hasattr(pltpu, "reciprocal"):
        return pltpu.reciprocal(v, approx=True)
    return 1.0 / v


def _silu(x):
    e = jnp.exp2(x * (-_LOG2E))
    return (x * _recip(1.0 + e)).astype(jnp.bfloat16)


_NW = 32
_PER_W = (2 * N_EDGES) // _NW


def _sc_gather_z(atomic_numbers, flat_idx):
    mesh = plsc.VectorSubcoreMesh(core_axis_name="c", subcore_axis_name="s")
    cp = pltpu.CompilerParams()
    if "needs_layout_passes" in pltpu.CompilerParams.__dataclass_fields__:
        cp = dataclasses.replace(cp, needs_layout_passes=False)

    @functools.partial(
        pl.kernel,
        mesh=mesh,
        compiler_params=cp,
        out_type=jax.ShapeDtypeStruct((2 * N_EDGES,), jnp.int32),
        scratch_types=[
            pltpu.VMEM((N_NODES,), jnp.int32),
            pltpu.VMEM((_PER_W,), jnp.int32),
            pltpu.VMEM((_PER_W,), jnp.int32),
        ],
    )
    def gather_kernel(tab_hbm, idx_hbm, out_hbm, tab_v, idx_v, out_v):
        wid = lax.axis_index("s") * 2 + lax.axis_index("c")
        base = wid * _PER_W
        pltpu.sync_copy(tab_hbm, tab_v)
        pltpu.sync_copy(idx_hbm.at[pl.ds(base, _PER_W)], idx_v)

        @pl.loop(0, _PER_W, step=16)
        def _(i):
            idx = idx_v[pl.ds(i, 16)]
            out_v[pl.ds(i, 16)] = plsc.load_gather(tab_v, [idx])

        pltpu.sync_copy(out_v, out_hbm.at[pl.ds(base, _PER_W)])

    return gather_kernel(atomic_numbers, flat_idx)


def _tc_body(d_ref, zs_ref, zt_ref, ti_ref, batch_ref, offs_ref, srange_ref,
             wf_ref, we_ref, wp_ref, bd_ref, be_ref, bp_ref,
             w1p_ref, b1p_ref, w2p_ref, b2p_ref,
             out_ref, acc_ref, cnt_ref, bounds_ref):
    i = pl.program_id(0)

    @pl.when(i == 0)
    def _():
        acc_ref[...] = jnp.zeros_like(acc_ref)
        cnt_ref[...] = jnp.zeros_like(cnt_ref)
        b = batch_ref[...]
        lanes = lax.broadcasted_iota(jnp.int32, (1, BATCH), 1)
        starts = jnp.sum((b < lanes).astype(jnp.int32), axis=0, keepdims=True)
        ends = jnp.sum((b <= lanes).astype(jnp.int32), axis=0, keepdims=True)
        starts_c = jnp.transpose(
            jnp.broadcast_to(starts, (BATCH, BATCH)))[:, 0:1]
        ends_c = jnp.transpose(jnp.broadcast_to(ends, (BATCH, BATCH)))[:, 0:1]
        bounds_ref[:, 0:1] = starts_c
        bounds_ref[:, 1:2] = ends_c

    d = d_ref[0]
    offs = offs_ref[...]
    diff = d - offs
    arg = (_COEFF2 * diff) * diff
    gauss = jnp.exp2(arg.astype(jnp.bfloat16))

    srange = srange_ref[...]
    ohs = (zs_ref[0] == srange).astype(jnp.bfloat16)
    oht = (zt_ref[0] == srange).astype(jnp.bfloat16)

    wf = wf_ref[...]
    x = (jnp.dot(wf[:, :NUM_GAUSS], gauss,
                 preferred_element_type=jnp.float32)
         + jnp.dot(wf[:, NUM_GAUSS:NUM_GAUSS + OH], ohs,
                   preferred_element_type=jnp.float32)
         + jnp.dot(wf[:, NUM_GAUSS + OH:], oht,
                   preferred_element_type=jnp.float32))
    x = _silu(x + bd_ref[...])
    x = jnp.dot(we_ref[...], x, preferred_element_type=jnp.float32)
    x = _silu(x + be_ref[...])
    x = jnp.dot(wp_ref[...], x, preferred_element_type=jnp.float32)
    x = _silu(x + bp_ref[...])

    ti = ti_ref[0]
    starts_c = bounds_ref[:, 0:1]
    ends_c = bounds_ref[:, 1:2]
    seg = jnp.logical_and(ti >= starts_c, ti < ends_c)
    segb = seg.astype(jnp.bfloat16)

    acc_ref[...] += lax.dot_general(
        x, segb, (((1,), (1,)), ((), ())), preferred_element_type=jnp.float32)
    ones_row = jnp.ones((1, BLK), jnp.bfloat16)
    cnt_ref[...] += lax.dot_general(
        ones_row, segb, (((1,), (1,)), ((), ())),
        preferred_element_type=jnp.float32)

    @pl.when(i == NSTEPS - 1)
    def _():
        xgt = acc_ref[...] / (cnt_ref[...] + 0.001)
        h = jnp.dot(w1p_ref[...], xgt.astype(jnp.bfloat16),
                    preferred_element_type=jnp.float32)
        h = _silu(h + b1p_ref[...])
        out = jnp.dot(w2p_ref[...], h,
                      preferred_element_type=jnp.float32)
        out = out + b2p_ref[...]
        out_ref[...] = jnp.transpose(out)


def _row_spec():
    return pl.BlockSpec((1, 1, BLK), lambda i: (i, 0, 0))


def _full_spec(shape):
    return pl.BlockSpec(shape, lambda i: tuple(0 for _ in shape))


def kernel(atomic_numbers, edge_distance, edge_index, batch, batch_size,
           source_emb, target_emb, W_dist, b_dist, W_edge, b_edge,
           W1_pre, b1_pre, W1_post, b1_post, W2_post, b2_post):
    flat_idx = edge_index.reshape(2 * N_EDGES)
    zz = _sc_gather_z(atomic_numbers, flat_idx)
    zs = zz[:N_EDGES].reshape(NSTEPS, 1, BLK)
    zt = zz[N_EDGES:].reshape(NSTEPS, 1, BLK)

    d = edge_distance.reshape(NSTEPS, 1, BLK)
    ti = edge_index[1].reshape(NSTEPS, 1, BLK)
    b2d = batch.reshape(N_NODES, 1)

    offs = jnp.linspace(0.0, 8.0, NUM_GAUSS).reshape(NUM_GAUSS, 1)
    srange = jnp.arange(OH, dtype=jnp.int32).reshape(OH, 1)

    pad = jnp.zeros((OH - MAX_ELEM, HIDDEN), jnp.float32)
    w_fused = jnp.concatenate(
        [W_dist, source_emb, pad, target_emb, pad],
        axis=0).T.astype(jnp.bfloat16)
    we = W_edge.T.astype(jnp.bfloat16)
    wp = W1_pre.T.astype(jnp.bfloat16)
    w1p = W1_post.T.astype(jnp.bfloat16)
    w2p = W2_post.T.astype(jnp.bfloat16)
    bd = b_dist.reshape(HIDDEN, 1)
    be = b_edge.reshape(HIDDEN, 1)
    bp = b1_pre.reshape(HIDDEN, 1)
    b1p = b1_post.reshape(HIDDEN, 1)
    b2p = b2_post.reshape(NUM_EXPERTS, 1)

    out = pl.pallas_call(
        _tc_body,
        grid=(NSTEPS,),
        in_specs=[
            _row_spec(),
            _row_spec(),
            _row_spec(),
            _row_spec(),
            _full_spec((N_NODES, 1)),
            _full_spec((NUM_GAUSS, 1)),
            _full_spec((OH, 1)),
            _full_spec((HIDDEN, NUM_GAUSS + 2 * OH)),
            _full_spec((HIDDEN, HIDDEN)),
            _full_spec((HIDDEN, HIDDEN)),
            _full_spec((HIDDEN, 1)),
            _full_spec((HIDDEN, 1)),
            _full_spec((HIDDEN, 1)),
            _full_spec((HIDDEN, HIDDEN)),
            _full_spec((HIDDEN, 1)),
            _full_spec((NUM_EXPERTS, HIDDEN)),
            _full_spec((NUM_EXPERTS, 1)),
        ],
        out_specs=_full_spec((BATCH, NUM_EXPERTS)),
        out_shape=jax.ShapeDtypeStruct((BATCH, NUM_EXPERTS), jnp.float32),
        scratch_shapes=[
            pltpu.VMEM((HIDDEN, BATCH), jnp.float32),
            pltpu.VMEM((1, BATCH), jnp.float32),
            pltpu.VMEM((BATCH, 8), jnp.int32),
        ],
        compiler_params=pltpu.CompilerParams(
            dimension_semantics=("arbitrary",)),
    )(d, zs, zt, ti, b2d, offs, srange, w_fused, we, wp, bd, be, bp,
      w1p, b1p, w2p, b2p)
    return out
